# baseline (speedup 1.0000x reference)
import functools

import jax
import jax.numpy as jnp
from jax import lax
from jax.experimental import pallas as pl
from jax.experimental.pallas import tpu as pltpu
from jax.experimental.pallas import tpu_sc as plsc

_LANES = 16
_IDX_W = 128


def _build_sc_call(B, F, VP):
    NW = 32
    BPW = B // NW
    KW = BPW // _IDX_W
    RCH = BPW // _LANES
    DEPTH = 8

    mesh = plsc.VectorSubcoreMesh(core_axis_name="c", subcore_axis_name="s")

    @functools.partial(
        pl.kernel,
        out_type=jax.ShapeDtypeStruct((B,), jnp.float32),
        mesh=mesh,
        scratch_types=[
            pltpu.VMEM((F, BPW), jnp.int32),
            pltpu.VMEM((F, BPW), jnp.float32),
            pltpu.VMEM((_LANES,), jnp.float32),
            pltpu.VMEM((BPW,), jnp.float32),
            pltpu.SemaphoreType.DMA((F,)),
            pltpu.SemaphoreType.DMA,
        ],
    )
    def sc_call(xt_hbm, fc_hbm, bias_hbm, out_hbm, idx_v, vals_v, bias_v,
                out_v, isem, gsem):
        wid = lax.axis_index("s") * 2 + lax.axis_index("c")
        base = wid * BPW

        for f in range(F):
            pltpu.async_copy(xt_hbm.at[f, pl.ds(base, BPW)],
                             idx_v.at[f], isem.at[f])
        pltpu.sync_copy(bias_hbm, bias_v)

        def _fire_row(f):
            pltpu.make_async_copy(xt_hbm.at[f, pl.ds(base, BPW)],
                                  idx_v.at[f], isem.at[f]).wait()
            for k in range(KW):
                sl = pl.ds(k * _IDX_W, _IDX_W)
                pltpu.async_copy(fc_hbm.at[idx_v.at[f, sl]],
                                 vals_v.at[f, sl], gsem)

        def _drain_row():
            for k in range(KW):
                sl = pl.ds(k * _IDX_W, _IDX_W)
                pltpu.make_async_copy(fc_hbm.at[idx_v.at[0, sl]],
                                      vals_v.at[0, sl], gsem).wait()

        for f in range(DEPTH - 1):
            _fire_row(f)

        @pl.loop(DEPTH - 1, F)
        def _gather_row(f):
            _fire_row(f)
            _drain_row()

        for _ in range(DEPTH - 1):
            _drain_row()

        @pl.loop(0, RCH)
        def _reduce(c):
            sl = pl.ds(c * _LANES, _LANES)
            acc = bias_v[...]
            for f in range(F):
                acc = acc + vals_v[f, sl]
            out_v[pl.ds(c * _LANES, _LANES)] = acc

        pltpu.sync_copy(out_v, out_hbm.at[pl.ds(base, BPW)])

    return sc_call


def kernel(x, fc, bias):
    B, F = x.shape
    V, OD = fc.shape
    xt = x.astype(jnp.int32).T
    vp = (V + 8 * _IDX_W - 1) // (8 * _IDX_W) * (8 * _IDX_W)
    fcf = jnp.pad(fc, ((0, vp - V), (0, 0))).reshape(vp)
    bias16 = jnp.broadcast_to(bias.astype(jnp.float32), (_LANES,))
    out = _build_sc_call(B, F, vp)(xt, fcf, bias16)
    return out.reshape(B, OD)


# speedup vs baseline: 1.0508x; 1.0508x over previous
import functools

import jax
import jax.numpy as jnp
from jax import lax
from jax.experimental import pallas as pl
from jax.experimental.pallas import tpu as pltpu
from jax.experimental.pallas import tpu_sc as plsc

_LANES = 16
_IDX_W = 128


def _build_sc_call(B, F, VP):
    NW = 32
    BPW = B // NW
    KW = BPW // _IDX_W
    VPB = _IDX_W // _LANES

    mesh = plsc.VectorSubcoreMesh(core_axis_name="c", subcore_axis_name="s")

    @functools.partial(
        pl.kernel,
        out_type=jax.ShapeDtypeStruct((B,), jnp.float32),
        mesh=mesh,
        scratch_types=[
            pltpu.VMEM((F, BPW), jnp.int32),
            pltpu.VMEM((F, BPW), jnp.float32),
            pltpu.VMEM((_LANES,), jnp.float32),
            pltpu.VMEM((BPW,), jnp.float32),
            pltpu.SemaphoreType.DMA((KW,)),
            pltpu.SemaphoreType.DMA,
        ],
    )
    def sc_call(xt_hbm, fc_hbm, bias_hbm, out_hbm, idx_v, vals_v, bias_v,
                out_v, isem, gsem):
        wid = lax.axis_index("s") * 2 + lax.axis_index("c")
        base = wid * BPW

        def _blk(k):
            return pl.ds(k * _IDX_W, _IDX_W)

        for k in range(KW):
            pltpu.async_copy(xt_hbm.at[:, pl.ds(base + k * _IDX_W, _IDX_W)],
                             idx_v.at[:, _blk(k)], isem.at[k])
        pltpu.sync_copy(bias_hbm, bias_v)

        def _fire_block(k):
            pltpu.make_async_copy(
                xt_hbm.at[:, pl.ds(base + k * _IDX_W, _IDX_W)],
                idx_v.at[:, _blk(k)], isem.at[k]).wait()
            for f in range(F):
                pltpu.async_copy(fc_hbm.at[idx_v.at[f, _blk(k)]],
                                 vals_v.at[f, _blk(k)], gsem)

        def _drain_block():
            for f in range(F):
                pltpu.make_async_copy(fc_hbm.at[idx_v.at[0, _blk(0)]],
                                      vals_v.at[0, _blk(0)], gsem).wait()

        def _reduce_block(k):
            for j in range(VPB):
                sl = pl.ds(k * _IDX_W + j * _LANES, _LANES)
                acc = bias_v[...]
                for f in range(F):
                    acc = acc + vals_v[f, sl]
                out_v[sl] = acc

        _fire_block(0)
        for k in range(1, KW):
            _fire_block(k)
            _drain_block()
            _reduce_block(k - 1)
        _drain_block()
        _reduce_block(KW - 1)

        pltpu.sync_copy(out_v, out_hbm.at[pl.ds(base, BPW)])

    return sc_call


def kernel(x, fc, bias):
    B, F = x.shape
    V, OD = fc.shape
    xt = x.astype(jnp.int32).T
    vp = (V + 8 * _IDX_W - 1) // (8 * _IDX_W) * (8 * _IDX_W)
    fcf = jnp.pad(fc, ((0, vp - V), (0, 0))).reshape(vp)
    bias16 = jnp.broadcast_to(bias.astype(jnp.float32), (_LANES,))
    out = _build_sc_call(B, F, vp)(xt, fcf, bias16)
    return out.reshape(B, OD)


# speedup vs baseline: 1.3637x; 1.2977x over previous
import functools

import jax
import jax.numpy as jnp
from jax import lax
from jax.experimental import pallas as pl
from jax.experimental.pallas import tpu as pltpu
from jax.experimental.pallas import tpu_sc as plsc

_LANES = 16
_IDX_W = 128


def _build_sc_call(B, F, VP):
    NW = 32
    BPW = B // NW
    KW = BPW // _IDX_W
    VPB = _IDX_W // _LANES

    mesh = plsc.VectorSubcoreMesh(core_axis_name="c", subcore_axis_name="s")

    @functools.partial(
        pl.kernel,
        out_type=jax.ShapeDtypeStruct((B,), jnp.float32),
        mesh=mesh,
        scratch_types=[
            pltpu.VMEM((F, BPW), jnp.int32),
            pltpu.VMEM((F, BPW), jnp.float32),
            pltpu.VMEM((_LANES,), jnp.float32),
            pltpu.VMEM((BPW,), jnp.float32),
            pltpu.SemaphoreType.DMA((KW,)),
            pltpu.SemaphoreType.DMA,
        ],
    )
    def sc_call(xt_hbm, fc2_hbm, bias_hbm, out_hbm, idx_v, vals_v, bias_v,
                out_v, isem, gsem):
        wid = lax.axis_index("s") * 2 + lax.axis_index("c")
        base = wid * BPW
        fc_hbm = fc2_hbm.at[0]

        def _blk(k):
            return pl.ds(k * _IDX_W, _IDX_W)

        for k in range(KW):
            pltpu.async_copy(xt_hbm.at[:, pl.ds(base + k * _IDX_W, _IDX_W)],
                             idx_v.at[:, _blk(k)], isem.at[k])
        pltpu.sync_copy(bias_hbm, bias_v)

        def _fire_block(k):
            pltpu.make_async_copy(
                xt_hbm.at[:, pl.ds(base + k * _IDX_W, _IDX_W)],
                idx_v.at[:, _blk(k)], isem.at[k]).wait()
            for f in range(F):
                pltpu.async_copy(fc_hbm.at[idx_v.at[f, _blk(k)]],
                                 vals_v.at[f, _blk(k)], gsem)

        def _drain_block():
            for f in range(F):
                pltpu.make_async_copy(fc_hbm.at[idx_v.at[0, _blk(0)]],
                                      vals_v.at[0, _blk(0)], gsem).wait()

        def _reduce_block(k):
            for j in range(VPB):
                sl = pl.ds(k * _IDX_W + j * _LANES, _LANES)
                acc = bias_v[...]
                for f in range(F):
                    acc = acc + vals_v[f, sl]
                out_v[sl] = acc

        _fire_block(0)
        for k in range(1, KW):
            _fire_block(k)
            _drain_block()
            _reduce_block(k - 1)
        _drain_block()
        _reduce_block(KW - 1)

        pltpu.sync_copy(out_v, out_hbm.at[pl.ds(base, BPW)])

    return sc_call


def kernel(x, fc, bias):
    B, F = x.shape
    V, OD = fc.shape
    xt = x.astype(jnp.int32).T
    fcf = fc.reshape(1, V)
    bias16 = jnp.broadcast_to(bias.astype(jnp.float32), (_LANES,))
    out = _build_sc_call(B, F, V)(xt, fcf, bias16)
    return out.reshape(B, OD)
